# speedup vs baseline: 1.6493x; 1.6493x over previous
import functools

import jax
import jax.numpy as jnp
from jax import lax
from jax.experimental import pallas as pl
from jax.experimental.pallas import tpu as pltpu
from jax.experimental.pallas import tpu_sc as plsc

NW = 32
C = 64


def _emb_body(idx_hbm, table_hbm, out_hbm, idx_v, rows_v, gsem):
    nch = idx_hbm.shape[1]
    wid = lax.axis_index("s") * 2 + lax.axis_index("c")
    per_w = nch * C
    base = wid * per_w
    pltpu.sync_copy(idx_hbm.at[wid], idx_v)

    def chunk(j, carry):
        pltpu.async_copy(table_hbm.at[idx_v.at[j]], rows_v, gsem).wait()
        row0 = pl.multiple_of(base + j * C, 8)
        pltpu.sync_copy(rows_v, out_hbm.at[pl.ds(row0, C)])
        return carry

    lax.fori_loop(0, nch, chunk, 0)


def kernel(input_ids, embed_table):
    b, s = input_ids.shape
    v, d = embed_table.shape
    ntok = b * s
    per_w = ntok // NW
    nch = per_w // C
    ids = input_ids.reshape(-1).astype(jnp.int32).reshape(NW, nch, C)

    run = functools.partial(
        pl.kernel,
        mesh=plsc.VectorSubcoreMesh(core_axis_name="c", subcore_axis_name="s"),
        out_type=jax.ShapeDtypeStruct((ntok, d), jnp.float32),
        scratch_types=[
            pltpu.VMEM((nch, C), jnp.int32),
            pltpu.VMEM((C, d), jnp.float32),
            pltpu.SemaphoreType.DMA,
        ],
    )(_emb_body)

    out = run(ids, embed_table)
    return out.reshape(b, s, d)


# speedup vs baseline: 1.8405x; 1.1159x over previous
import functools

import jax
import jax.numpy as jnp
from jax import lax
from jax.experimental import pallas as pl
from jax.experimental.pallas import tpu as pltpu
from jax.experimental.pallas import tpu_sc as plsc

NW = 32
C = 64


def _emb_body(idx_hbm, table_hbm, out_hbm, idx_v, rows_a, rows_b,
              gsa, gsb, wsa, wsb):
    nch = idx_hbm.shape[1]
    wid = lax.axis_index("s") * 2 + lax.axis_index("c")
    per_w = nch * C
    base = wid * per_w
    pltpu.sync_copy(idx_hbm.at[wid], idx_v)

    def out_slab(j):
        return out_hbm.at[pl.ds(pl.multiple_of(base + j * C, 8), C)]

    def start_gather(j, rows, sem):
        pltpu.async_copy(table_hbm.at[idx_v.at[j]], rows, sem)

    def wait_gather(rows, sem):
        pltpu.make_async_copy(table_hbm.at[idx_v.at[0]], rows, sem).wait()

    def start_write(j, rows, sem):
        pltpu.async_copy(rows, out_slab(j), sem)

    def wait_write(rows, sem):
        pltpu.make_async_copy(rows, out_slab(0), sem).wait()

    start_gather(0, rows_a, gsa)
    start_gather(1, rows_b, gsb)

    def pair(jj, carry):
        j0 = jj * 2
        wait_gather(rows_a, gsa)
        start_write(j0, rows_a, wsa)
        wait_gather(rows_b, gsb)
        start_write(j0 + 1, rows_b, wsb)
        wait_write(rows_a, wsa)
        start_gather(j0 + 2, rows_a, gsa)
        wait_write(rows_b, wsb)
        start_gather(j0 + 3, rows_b, gsb)
        return carry

    lax.fori_loop(0, nch // 2 - 1, pair, 0)

    j0 = nch - 2
    wait_gather(rows_a, gsa)
    start_write(j0, rows_a, wsa)
    wait_gather(rows_b, gsb)
    start_write(j0 + 1, rows_b, wsb)
    wait_write(rows_a, wsa)
    wait_write(rows_b, wsb)


def kernel(input_ids, embed_table):
    b, s = input_ids.shape
    v, d = embed_table.shape
    ntok = b * s
    per_w = ntok // NW
    nch = per_w // C
    ids = input_ids.reshape(-1).astype(jnp.int32).reshape(NW, nch, C)

    run = functools.partial(
        pl.kernel,
        mesh=plsc.VectorSubcoreMesh(core_axis_name="c", subcore_axis_name="s"),
        out_type=jax.ShapeDtypeStruct((ntok, d), jnp.float32),
        scratch_types=[
            pltpu.VMEM((nch, C), jnp.int32),
            pltpu.VMEM((C, d), jnp.float32),
            pltpu.VMEM((C, d), jnp.float32),
            pltpu.SemaphoreType.DMA,
            pltpu.SemaphoreType.DMA,
            pltpu.SemaphoreType.DMA,
            pltpu.SemaphoreType.DMA,
        ],
    )(_emb_body)

    out = run(ids, embed_table)
    return out.reshape(b, s, d)
